# baseline (speedup 1.0000x reference)
import jax
import jax.numpy as jnp
from jax.experimental import pallas as pl
from jax.experimental.pallas import tpu as pltpu

POSITIVE = 1e-6
K_CONF = 100
K_IOU = 50
N = 20000
PAD_N = 20480
NR = 160
NL = 128
GW = 32
GH = 32
GPIX = GW * GH
B = 4
BPS = 4
CHUNK = 8


def _nms_kernel(c0_ref, c1_ref, pk_ref, out_ref, gauss_ref, rows_ref,
                p_ref):
    f32 = jnp.float32
    lin3 = (jax.lax.broadcasted_iota(jnp.int32, (1, NR, NL), 1) * NL
            + jax.lax.broadcasted_iota(jnp.int32, (1, NR, NL), 2))
    biota = jax.lax.broadcasted_iota(jnp.int32, (BPS, 1, 1), 0)
    lane3 = jax.lax.broadcasted_iota(jnp.int32, (1, 1, NL), 2)

    c0 = c0_ref[:, :, :]
    c1 = c1_ref[:, :, :]
    m = jnp.maximum(c0, c1)
    e0 = jnp.exp(c0 - m)
    e1 = jnp.exp(c1 - m)
    p = jnp.where(lin3 < N, e1 / (e0 + e1), -1.0)

    rows_ref[:, :, :] = jnp.zeros((BPS, 128, 48), f32)
    p_ref[:, :, :] = p

    NG = NR // 8
    giota = jax.lax.broadcasted_iota(jnp.int32, (1, NG, 1), 1)
    lin8 = (jax.lax.broadcasted_iota(jnp.int32, (8, NL), 0) * NL
            + jax.lax.broadcasted_iota(jnp.int32, (8, NL), 1))
    sup0 = jnp.max(p.reshape(BPS, NG, 8, NL), axis=2)

    def body1(t, carry):
        sup, cvec = carry
        gm = jnp.max(jnp.max(sup, axis=2, keepdims=True), axis=1,
                     keepdims=True)
        gcand = jnp.where(sup == gm, giota, NG)
        gmin = jnp.min(jnp.min(gcand, axis=2, keepdims=True), axis=1,
                       keepdims=True)
        cvec = jnp.where(lane3 == t, gm, cvec)
        for b in range(BPS):
            g_b = jnp.sum(jnp.where(biota == b, gmin, 0))
            gm_b = jnp.sum(jnp.where(biota == b, gm, 0.0))
            pg = p_ref[b, pl.ds(g_b * 8, 8), :]
            loc_idx = jnp.min(jnp.where(pg == gm_b, lin8, 8 * NL))
            idx_b = g_b * (8 * NL) + loc_idx
            pg = jnp.where(lin8 == loc_idx, -jnp.inf, pg)
            p_ref[b, pl.ds(g_b * 8, 8), :] = pg
            newsup = jnp.max(pg, axis=0, keepdims=True)[None]
            sup = jnp.where((biota == b) & (giota == g_b), newsup, sup)
            pr = pk_ref[b, pl.ds(idx_b, 1), :]
            rows_ref[b, pl.ds(t, 1), 0:40] = pr
        return sup, cvec

    cvec0 = jnp.zeros((BPS, 1, NL), f32)
    _, cvec = jax.lax.fori_loop(0, K_CONF, body1, (sup0, cvec0))

    gx_pos = (jax.lax.broadcasted_iota(jnp.int32, (1, 1, GPIX), 2) % GW
              ).astype(f32) * (1.0 / (GW - 1))
    gy_pos = (jax.lax.broadcasted_iota(jnp.int32, (1, 1, GPIX), 2) // GW
              ).astype(f32) * (1.0 / (GH - 1))
    l0 = rows_ref[:, :, 0:1]
    l1 = rows_ref[:, :, 1:2]
    l2 = rows_ref[:, :, 2:3]
    l3 = rows_ref[:, :, 3:4]
    p0 = rows_ref[:, :, 36:37]
    p1 = rows_ref[:, :, 37:38]
    p2 = rows_ref[:, :, 38:39]
    p3 = rows_ref[:, :, 39:40]
    w = p2 * jnp.exp(l2 * 0.2)
    h = p3 * jnp.exp(l3 * 0.2)
    cx = p0 + l0 * 0.1 * p2
    cy = p1 + l1 * 0.1 * p3
    rdx = 1.0 / (2.0 * (w * 0.5) ** 2 + POSITIVE)
    rdy = 1.0 / (2.0 * (h * 0.5) ** 2 + POSITIVE)
    g = jnp.exp(-((gx_pos - cx) ** 2 * rdx + (gy_pos - cy) ** 2 * rdy))
    gauss_ref[:, :, :] = g
    S = jnp.sum(g, axis=2, keepdims=True)

    im = jnp.where(lane3 < K_CONF, jnp.zeros((BPS, 1, NL), f32), jnp.inf)
    for c in range(K_CONF // CHUNK + 1):
        j_lo = c * CHUNK
        n_cols = min(CHUNK, K_CONF - j_lo)
        if n_cols <= 0:
            break
        rc = min(128, (c + 1) * CHUNK)
        Gc = gauss_ref[:, 0:rc, :]
        Sc = S[:, 0:rc, :]
        rowc = jax.lax.broadcasted_iota(jnp.int32, (1, rc, 1), 1)

        def body2(jj, im, j_lo=j_lo, Gc=Gc, Sc=Sc, rowc=rowc):
            j = j_lo + jj
            gj = gauss_ref[:, pl.ds(j, 1), :]
            inter = jnp.sum(jnp.minimum(Gc, gj), axis=2, keepdims=True)
            sj = jnp.sum(gj, axis=2, keepdims=True)
            union = Sc + sj - inter
            iou = inter / (union + POSITIVE)
            masked = jnp.where(rowc < j, iou, 0.0)
            colmax = jnp.max(jnp.max(masked, axis=2, keepdims=True), axis=1,
                             keepdims=True)
            return jnp.where(lane3 == j, colmax, im)

        im = jax.lax.fori_loop(0, n_cols, body2, im)

    def body3(t, im):
        mvals = jnp.min(jnp.min(im, axis=2, keepdims=True), axis=1,
                        keepdims=True)
        cand = jnp.where(im == mvals, lane3, NL)
        kidx = jnp.min(jnp.min(cand, axis=2, keepdims=True), axis=1,
                       keepdims=True)
        for b in range(BPS):
            k_b = jnp.sum(jnp.where(biota == b, kidx, 0))
            c_b = jnp.sum(jnp.where((lane3 == kidx) & (biota == b), cvec,
                                    0.0))
            row = rows_ref[b, pl.ds(k_b, 1), :]
            out_ref[b, pl.ds(t, 1), :] = jnp.concatenate(
                [row[:, 0:36], jnp.full((1, 1), c_b, jnp.float32),
                 row[:, 36:40]], axis=1)
        return jnp.where(lane3 == kidx, jnp.inf, im)

    jax.lax.fori_loop(0, K_IOU, body3, im)


def kernel(original, conf, loc, mask, priors):
    del original
    cp = jnp.pad(conf, ((0, 0), (0, PAD_N - N), (0, 0)))
    cp = cp.reshape(B, NR, NL, 2)
    c0 = cp[..., 0]
    c1 = cp[..., 1]
    priors_b = jnp.broadcast_to(priors[None], (B,) + priors.shape)
    packed = jnp.concatenate([loc, mask, priors_b], axis=2)
    return pl.pallas_call(
        _nms_kernel,
        grid=(B // BPS,),
        in_specs=[
            pl.BlockSpec((BPS, NR, NL), lambda i: (i, 0, 0)),
            pl.BlockSpec((BPS, NR, NL), lambda i: (i, 0, 0)),
            pl.BlockSpec((BPS, N, 40), lambda i: (i, 0, 0)),
        ],
        out_specs=pl.BlockSpec((BPS, K_IOU, 41), lambda i: (i, 0, 0)),
        out_shape=jax.ShapeDtypeStruct((B, K_IOU, 41), jnp.float32),
        scratch_shapes=[
            pltpu.VMEM((BPS, 128, GPIX), jnp.float32),
            pltpu.VMEM((BPS, 128, 48), jnp.float32),
            pltpu.VMEM((BPS, NR, NL), jnp.float32),
        ],
    )(c0, c1, packed)


# speedup vs baseline: 1.4224x; 1.4224x over previous
import jax
import jax.numpy as jnp
from jax.experimental import pallas as pl
from jax.experimental.pallas import tpu as pltpu

POSITIVE = 1e-6
K_CONF = 100
K_IOU = 50
N = 20000
PAD_N = 20480
NR = 160
NL = 128
GW = 32
GH = 32
GPIX = GW * GH
B = 4
BPS = 4
CHUNK = 8


def _nms_kernel(c0_ref, c1_ref, pk_ref, out_ref, gauss_ref, rows_ref):
    f32 = jnp.float32
    lin3 = (jax.lax.broadcasted_iota(jnp.int32, (1, NR, NL), 1) * NL
            + jax.lax.broadcasted_iota(jnp.int32, (1, NR, NL), 2))
    biota = jax.lax.broadcasted_iota(jnp.int32, (BPS, 1, 1), 0)
    lane3 = jax.lax.broadcasted_iota(jnp.int32, (1, 1, NL), 2)

    c0 = c0_ref[:, :, :]
    c1 = c1_ref[:, :, :]
    m = jnp.maximum(c0, c1)
    e0 = jnp.exp(c0 - m)
    e1 = jnp.exp(c1 - m)
    p = jnp.where(lin3 < N, e1 / (e0 + e1), -1.0)

    rows_ref[:, :, :] = jnp.zeros((BPS, 128, 48), f32)

    def body1(t, carry):
        p, cvec = carry
        mvals = jnp.max(jnp.max(p, axis=2, keepdims=True), axis=1,
                        keepdims=True)
        cand = jnp.where(p == mvals, lin3, PAD_N)
        idxs = jnp.min(jnp.min(cand, axis=2, keepdims=True), axis=1,
                       keepdims=True)
        cvec = jnp.where(lane3 == t, mvals, cvec)
        for b in range(BPS):
            idx_b = jnp.sum(jnp.where(biota == b, idxs, 0))
            pr = pk_ref[b, pl.ds(idx_b, 1), :]
            rows_ref[b, pl.ds(t, 1), 0:40] = pr
        p = jnp.where(lin3 == idxs, -jnp.inf, p)
        return p, cvec

    cvec0 = jnp.zeros((BPS, 1, NL), f32)
    _, cvec = jax.lax.fori_loop(0, K_CONF, body1, (p, cvec0))

    gx_pos = (jax.lax.broadcasted_iota(jnp.int32, (1, 1, GPIX), 2) % GW
              ).astype(f32) * (1.0 / (GW - 1))
    gy_pos = (jax.lax.broadcasted_iota(jnp.int32, (1, 1, GPIX), 2) // GW
              ).astype(f32) * (1.0 / (GH - 1))
    l0 = rows_ref[:, :, 0:1]
    l1 = rows_ref[:, :, 1:2]
    l2 = rows_ref[:, :, 2:3]
    l3 = rows_ref[:, :, 3:4]
    p0 = rows_ref[:, :, 36:37]
    p1 = rows_ref[:, :, 37:38]
    p2 = rows_ref[:, :, 38:39]
    p3 = rows_ref[:, :, 39:40]
    w = p2 * jnp.exp(l2 * 0.2)
    h = p3 * jnp.exp(l3 * 0.2)
    cx = p0 + l0 * 0.1 * p2
    cy = p1 + l1 * 0.1 * p3
    rdx = 1.0 / (2.0 * (w * 0.5) ** 2 + POSITIVE)
    rdy = 1.0 / (2.0 * (h * 0.5) ** 2 + POSITIVE)
    g = jnp.exp(-((gx_pos - cx) ** 2 * rdx + (gy_pos - cy) ** 2 * rdy))
    gauss_ref[:, :, :] = g
    S = jnp.sum(g, axis=2, keepdims=True)

    im = jnp.where(lane3 < K_CONF, jnp.zeros((BPS, 1, NL), f32), jnp.inf)
    for c in range(K_CONF // CHUNK + 1):
        j_lo = c * CHUNK
        n_cols = min(CHUNK, K_CONF - j_lo)
        if n_cols <= 0:
            break
        rc = min(128, (c + 1) * CHUNK)
        Gc = gauss_ref[:, 0:rc, :]
        Sc = S[:, 0:rc, :]
        rowc = jax.lax.broadcasted_iota(jnp.int32, (1, rc, 1), 1)

        def body2(jj, im, j_lo=j_lo, Gc=Gc, Sc=Sc, rowc=rowc):
            j = j_lo + jj
            gj = gauss_ref[:, pl.ds(j, 1), :]
            inter = jnp.sum(jnp.minimum(Gc, gj), axis=2, keepdims=True)
            sj = jnp.sum(gj, axis=2, keepdims=True)
            union = Sc + sj - inter
            iou = inter / (union + POSITIVE)
            masked = jnp.where(rowc < j, iou, 0.0)
            colmax = jnp.max(jnp.max(masked, axis=2, keepdims=True), axis=1,
                             keepdims=True)
            return jnp.where(lane3 == j, colmax, im)

        im = jax.lax.fori_loop(0, n_cols, body2, im)

    def body3(t, im):
        mvals = jnp.min(jnp.min(im, axis=2, keepdims=True), axis=1,
                        keepdims=True)
        cand = jnp.where(im == mvals, lane3, NL)
        kidx = jnp.min(jnp.min(cand, axis=2, keepdims=True), axis=1,
                       keepdims=True)
        for b in range(BPS):
            k_b = jnp.sum(jnp.where(biota == b, kidx, 0))
            c_b = jnp.sum(jnp.where((lane3 == kidx) & (biota == b), cvec,
                                    0.0))
            row = rows_ref[b, pl.ds(k_b, 1), :]
            out_ref[b, pl.ds(t, 1), :] = jnp.concatenate(
                [row[:, 0:36], jnp.full((1, 1), c_b, jnp.float32),
                 row[:, 36:40]], axis=1)
        return jnp.where(lane3 == kidx, jnp.inf, im)

    jax.lax.fori_loop(0, K_IOU, body3, im)


def kernel(original, conf, loc, mask, priors):
    del original
    cp = jnp.pad(conf, ((0, 0), (0, PAD_N - N), (0, 0)))
    cp = cp.reshape(B, NR, NL, 2)
    c0 = cp[..., 0]
    c1 = cp[..., 1]
    priors_b = jnp.broadcast_to(priors[None], (B,) + priors.shape)
    packed = jnp.concatenate([loc, mask, priors_b], axis=2)
    return pl.pallas_call(
        _nms_kernel,
        grid=(B // BPS,),
        in_specs=[
            pl.BlockSpec((BPS, NR, NL), lambda i: (i, 0, 0)),
            pl.BlockSpec((BPS, NR, NL), lambda i: (i, 0, 0)),
            pl.BlockSpec((BPS, N, 40), lambda i: (i, 0, 0)),
        ],
        out_specs=pl.BlockSpec((BPS, K_IOU, 41), lambda i: (i, 0, 0)),
        out_shape=jax.ShapeDtypeStruct((B, K_IOU, 41), jnp.float32),
        scratch_shapes=[
            pltpu.VMEM((BPS, 128, GPIX), jnp.float32),
            pltpu.VMEM((BPS, 128, 48), jnp.float32),
        ],
    )(c0, c1, packed)
